# baseline (speedup 1.0000x reference)
import jax
import jax.numpy as jnp
import numpy as np
from jax import lax
from jax.experimental import pallas as pl
from jax.experimental.pallas import tpu as pltpu

BN_EPS = 1e-5
VMEM_LIMIT_BYTES = 48 << 20
RDEPTH = 4
WDEPTH = 2


def _make_copy_stats_kernel(n, cin, sup):
    nsup = n // sup

    def copy_stats_kernel(x_hbm, o_hbm, psum_ref, psq_ref,
                          xbuf, wbuf, rsem, wsem, acc_s, acc_q):
        def read(s):
            return pltpu.make_async_copy(
                x_hbm.at[pl.ds(s * sup, sup)], xbuf.at[s % RDEPTH],
                rsem.at[s % RDEPTH])

        def write(s):
            return pltpu.make_async_copy(
                wbuf.at[s % WDEPTH],
                o_hbm.at[pl.ds(s * sup, sup), pl.ds(0, cin)],
                wsem.at[s % WDEPTH])

        for s in range(min(RDEPTH, nsup)):
            read(s).start()
        for s in range(nsup):
            read(s).wait()
            xc = xbuf[s % RDEPTH]
            xs = jnp.sum(xc, axis=0)
            xq = jnp.sum(xc * xc, axis=0)
            ps = jnp.sum(xs, axis=1, keepdims=True)
            pq = jnp.sum(xq, axis=1, keepdims=True)
            if s == 0:
                acc_s[...] = ps
                acc_q[...] = pq
            else:
                acc_s[...] += ps
                acc_q[...] += pq
            if s >= WDEPTH:
                write(s - WDEPTH).wait()
            wbuf[s % WDEPTH] = xc
            write(s).start()
            if s + RDEPTH < nsup:
                read(s + RDEPTH).start()
        for s in range(max(nsup - WDEPTH, 0), nsup):
            write(s).wait()
        psum_ref[...] = acc_s[...]
        psq_ref[...] = acc_q[...]

    return copy_stats_kernel


def _make_conv_kernel(n, cin, cout, h, w, inv_count, sup):
    hw = h * w
    nsup = n // sup

    def conv_kernel(x_hbm, psum_ref, psq_ref, gamma_ref, beta_ref, w_ref,
                    o_prev_hbm, o_hbm, xbuf, obuf, rsem, wsem):
        mean = psum_ref[...] * inv_count
        var = psq_ref[...] * inv_count - mean * mean
        inv_std = lax.rsqrt(var + BN_EPS)
        scale = gamma_ref[...] * inv_std
        shift = beta_ref[...] - mean * scale

        pos = lax.broadcasted_iota(jnp.int32, (1, hw), 1)
        col = pos % w
        row = pos // w
        col_ok = {-1: col >= 1, 1: col < (w - 1)}
        row_ok = {-1: row >= 1, 1: row < (h - 1)}
        taps = []
        for kh in range(3):
            for kw in range(3):
                dh, dw = kh - 1, kw - 1
                m = None
                if dh != 0:
                    m = row_ok[dh]
                if dw != 0:
                    m = col_ok[dw] if m is None else jnp.logical_and(m, col_ok[dw])
                taps.append((kh * 3 + kw, dh * w + dw, m))

        wstk = w_ref[...]

        def read(s):
            return pltpu.make_async_copy(
                x_hbm.at[pl.ds(s * sup, sup)], xbuf.at[s % RDEPTH],
                rsem.at[s % RDEPTH])

        def write(s):
            return pltpu.make_async_copy(
                obuf.at[s % WDEPTH],
                o_hbm.at[pl.ds(s * sup, sup), pl.ds(cin, cout)],
                wsem.at[s % WDEPTH])

        for s in range(min(RDEPTH, nsup)):
            read(s).start()
        for s in range(nsup):
            read(s).wait()
            if s >= WDEPTH:
                write(s - WDEPTH).wait()
            rslot, oslot = s % RDEPTH, s % WDEPTH
            for b in range(sup):
                xb = xbuf[rslot, b]
                yb = jnp.maximum(xb * scale + shift, 0.0)
                z = jnp.dot(wstk, yb, preferred_element_type=jnp.float32)
                acc = None
                for k, soff, m in taps:
                    zk = z[k * cout:(k + 1) * cout, :]
                    if soff != 0:
                        zk = pltpu.roll(zk, (-soff) % hw, 1)
                    if m is not None:
                        zk = jnp.where(m, zk, 0.0)
                    acc = zk if acc is None else acc + zk
                obuf[oslot, b] = acc
            write(s).start()
            if s + RDEPTH < nsup:
                read(s + RDEPTH).start()
        for s in range(max(nsup - WDEPTH, 0), nsup):
            write(s).wait()

    return conv_kernel


def kernel(x, conv_w, gamma, beta):
    n, cin, h, w = x.shape
    cout = conv_w.shape[0]
    hw = h * w
    ctot = cin + cout

    x3 = x.reshape(n, cin, hw)
    g2 = gamma.reshape(cin, 1).astype(jnp.float32)
    b2 = beta.reshape(cin, 1).astype(jnp.float32)
    wstk = jnp.transpose(conv_w, (2, 3, 0, 1)).reshape(9 * cout, cin).astype(x.dtype)

    sup = max(d for d in (8, 4, 2, 1) if n % d == 0)

    out_a, psum, psq = pl.pallas_call(
        _make_copy_stats_kernel(n, cin, sup),
        out_shape=(jax.ShapeDtypeStruct((n, ctot, hw), x.dtype),
                   jax.ShapeDtypeStruct((cin, 1), jnp.float32),
                   jax.ShapeDtypeStruct((cin, 1), jnp.float32)),
        in_specs=[pl.BlockSpec(memory_space=pl.ANY)],
        out_specs=(pl.BlockSpec(memory_space=pl.ANY),
                   pl.BlockSpec((cin, 1), lambda: (0, 0)),
                   pl.BlockSpec((cin, 1), lambda: (0, 0))),
        scratch_shapes=[
            pltpu.VMEM((RDEPTH, sup, cin, hw), jnp.float32),
            pltpu.VMEM((WDEPTH, sup, cin, hw), jnp.float32),
            pltpu.SemaphoreType.DMA((RDEPTH,)),
            pltpu.SemaphoreType.DMA((WDEPTH,)),
            pltpu.VMEM((cin, 1), jnp.float32),
            pltpu.VMEM((cin, 1), jnp.float32),
        ],
        compiler_params=pltpu.CompilerParams(
            vmem_limit_bytes=VMEM_LIMIT_BYTES),
    )(x3)

    out3 = pl.pallas_call(
        _make_conv_kernel(n, cin, cout, h, w, 1.0 / float(n * hw), sup),
        out_shape=jax.ShapeDtypeStruct((n, ctot, hw), x.dtype),
        in_specs=[
            pl.BlockSpec(memory_space=pl.ANY),
            pl.BlockSpec((cin, 1), lambda: (0, 0)),
            pl.BlockSpec((cin, 1), lambda: (0, 0)),
            pl.BlockSpec((cin, 1), lambda: (0, 0)),
            pl.BlockSpec((cin, 1), lambda: (0, 0)),
            pl.BlockSpec((9 * cout, cin), lambda: (0, 0)),
            pl.BlockSpec(memory_space=pl.ANY),
        ],
        out_specs=pl.BlockSpec(memory_space=pl.ANY),
        input_output_aliases={6: 0},
        scratch_shapes=[
            pltpu.VMEM((RDEPTH, sup, cin, hw), jnp.float32),
            pltpu.VMEM((WDEPTH, sup, cout, hw), jnp.float32),
            pltpu.SemaphoreType.DMA((RDEPTH,)),
            pltpu.SemaphoreType.DMA((WDEPTH,)),
        ],
        compiler_params=pltpu.CompilerParams(
            vmem_limit_bytes=VMEM_LIMIT_BYTES),
    )(x3, psum, psq, g2, b2, wstk, out_a)

    return out3.reshape(n, ctot, h, w)


# speedup vs baseline: 1.0905x; 1.0905x over previous
import jax
import jax.numpy as jnp
import numpy as np
from jax import lax
from jax.experimental import pallas as pl
from jax.experimental.pallas import tpu as pltpu

BN_EPS = 1e-5
VMEM_LIMIT_BYTES = 48 << 20
STATS_BLOCK = 16
RDEPTH = 3
WDEPTH = 2


def _stats_kernel(x_ref, sum_ref, sq_ref):
    x = x_ref[...]
    xs = jnp.sum(x, axis=0)
    xq = jnp.sum(x * x, axis=0)
    sum_ref[0] = jnp.sum(xs, axis=1, keepdims=True)
    sq_ref[0] = jnp.sum(xq, axis=1, keepdims=True)


def _make_main_kernel(n, cin, cout, h, w, inv_count, sup):
    hw = h * w
    nsup = n // sup

    def main_kernel(x_hbm, psum_ref, psq_ref, gamma_ref, beta_ref, w_ref,
                    o_hbm, xbuf, obuf, rsem, csem, wsem):
        s0 = jnp.sum(psum_ref[...], axis=0)
        q0 = jnp.sum(psq_ref[...], axis=0)
        mean = s0 * inv_count
        var = q0 * inv_count - mean * mean
        inv_std = lax.rsqrt(var + BN_EPS)
        scale = gamma_ref[...] * inv_std
        shift = beta_ref[...] - mean * scale

        pos = lax.broadcasted_iota(jnp.int32, (1, hw), 1)
        col = pos % w
        row = pos // w
        col_ok = {-1: col >= 1, 1: col < (w - 1)}
        row_ok = {-1: row >= 1, 1: row < (h - 1)}
        taps = []
        for kh in range(3):
            for kw in range(3):
                dh, dw = kh - 1, kw - 1
                m = None
                if dh != 0:
                    m = row_ok[dh]
                if dw != 0:
                    m = col_ok[dw] if m is None else jnp.logical_and(m, col_ok[dw])
                taps.append((kh * 3 + kw, dh * w + dw, m))

        wstk = w_ref[...]

        def read(s):
            return pltpu.make_async_copy(
                x_hbm.at[pl.ds(s * sup, sup)], xbuf.at[s % RDEPTH],
                rsem.at[s % RDEPTH])

        def copy_out(s):
            return pltpu.make_async_copy(
                xbuf.at[s % RDEPTH],
                o_hbm.at[pl.ds(s * sup, sup), pl.ds(0, cin)],
                csem.at[s % RDEPTH])

        def write(s):
            return pltpu.make_async_copy(
                obuf.at[s % WDEPTH],
                o_hbm.at[pl.ds(s * sup, sup), pl.ds(cin, cout)],
                wsem.at[s % WDEPTH])

        for s in range(min(RDEPTH, nsup)):
            read(s).start()
        for s in range(nsup):
            read(s).wait()
            copy_out(s).start()
            if s >= WDEPTH:
                write(s - WDEPTH).wait()
            rslot, oslot = s % RDEPTH, s % WDEPTH
            for b in range(sup):
                xb = xbuf[rslot, b]
                yb = jnp.maximum(xb * scale + shift, 0.0)
                z = jnp.dot(wstk, yb, preferred_element_type=jnp.float32)
                acc = None
                for k, soff, m in taps:
                    zk = z[k * cout:(k + 1) * cout, :]
                    if soff != 0:
                        zk = pltpu.roll(zk, (-soff) % hw, 1)
                    if m is not None:
                        zk = jnp.where(m, zk, 0.0)
                    acc = zk if acc is None else acc + zk
                obuf[oslot, b] = acc
            write(s).start()
            if s + RDEPTH < nsup:
                copy_out(s).wait()
                read(s + RDEPTH).start()
        for s in range(max(nsup - RDEPTH, 0), nsup):
            copy_out(s).wait()
        for s in range(max(nsup - WDEPTH, 0), nsup):
            write(s).wait()

    return main_kernel


def kernel(x, conv_w, gamma, beta):
    n, cin, h, w = x.shape
    cout = conv_w.shape[0]
    hw = h * w
    ctot = cin + cout

    x3 = x.reshape(n, cin, hw)
    g2 = gamma.reshape(cin, 1).astype(jnp.float32)
    b2 = beta.reshape(cin, 1).astype(jnp.float32)
    wstk = jnp.transpose(conv_w, (2, 3, 0, 1)).reshape(9 * cout, cin).astype(x.dtype)

    sup = max(d for d in (8, 4, 2, 1) if n % d == 0)
    sb = STATS_BLOCK if n % STATS_BLOCK == 0 else 1
    nsteps = n // sb
    psum, psq = pl.pallas_call(
        _stats_kernel,
        out_shape=(jax.ShapeDtypeStruct((nsteps, cin, 1), jnp.float32),
                   jax.ShapeDtypeStruct((nsteps, cin, 1), jnp.float32)),
        grid=(nsteps,),
        in_specs=[pl.BlockSpec((sb, cin, hw), lambda i: (i, 0, 0))],
        out_specs=(pl.BlockSpec((1, cin, 1), lambda i: (i, 0, 0)),
                   pl.BlockSpec((1, cin, 1), lambda i: (i, 0, 0))),
        compiler_params=pltpu.CompilerParams(
            dimension_semantics=("parallel",),
            vmem_limit_bytes=VMEM_LIMIT_BYTES),
    )(x3)

    out3 = pl.pallas_call(
        _make_main_kernel(n, cin, cout, h, w, 1.0 / float(n * hw), sup),
        out_shape=jax.ShapeDtypeStruct((n, ctot, hw), x.dtype),
        in_specs=[
            pl.BlockSpec(memory_space=pl.ANY),
            pl.BlockSpec((nsteps, cin, 1), lambda: (0, 0, 0)),
            pl.BlockSpec((nsteps, cin, 1), lambda: (0, 0, 0)),
            pl.BlockSpec((cin, 1), lambda: (0, 0)),
            pl.BlockSpec((cin, 1), lambda: (0, 0)),
            pl.BlockSpec((9 * cout, cin), lambda: (0, 0)),
        ],
        out_specs=pl.BlockSpec(memory_space=pl.ANY),
        scratch_shapes=[
            pltpu.VMEM((RDEPTH, sup, cin, hw), jnp.float32),
            pltpu.VMEM((WDEPTH, sup, cout, hw), jnp.float32),
            pltpu.SemaphoreType.DMA((RDEPTH,)),
            pltpu.SemaphoreType.DMA((RDEPTH,)),
            pltpu.SemaphoreType.DMA((WDEPTH,)),
        ],
        compiler_params=pltpu.CompilerParams(
            vmem_limit_bytes=VMEM_LIMIT_BYTES),
    )(x3, psum, psq, g2, b2, wstk)

    return out3.reshape(n, ctot, h, w)
